# baseline (speedup 1.0000x reference)
import functools

import jax
import jax.numpy as jnp
from jax import lax
from jax.experimental import pallas as pl
from jax.experimental.pallas import tpu as pltpu
from jax.experimental.pallas import tpu_sc as plsc

D = 64
L = 16
NC = 2
NS = 16
NW = NC * NS
BCHUNK = 4


def _shuf(x, idx):
    dn = lax.GatherDimensionNumbers(
        offset_dims=(), collapsed_slice_dims=(0,), start_index_map=(0,))
    return lax.gather(x, idx[:, None], dn, (1,),
                      mode=lax.GatherScatterMode.PROMISE_IN_BOUNDS)


def _allsum(x):
    idx = lax.iota(jnp.int32, L)
    for d in (8, 4, 2, 1):
        x = x + _shuf(x, jnp.bitwise_xor(idx, d))
    return x


def _rsqrt(x):
    bits = plsc.bitcast(x, jnp.int32)
    bits = jnp.int32(0x5F3759DF) - lax.shift_right_logical(bits, 1)
    y = plsc.bitcast(bits, jnp.float32)
    for _ in range(2):
        y = y * (1.5 - 0.5 * x * y * y)
    return y


def _layernorm_chunk(rows_v, out_v, n, g_vecs, b_vecs):
    def row_body(r, c):
        vs = [rows_v[r, pl.ds(L * i, L)] for i in range(D // L)]
        t = (vs[0] + vs[1]) + (vs[2] + vs[3])
        q = (vs[0] * vs[0] + vs[1] * vs[1]) + (vs[2] * vs[2] + vs[3] * vs[3])
        mean_v = _allsum(t) * (1.0 / D)
        var_v = _allsum(q) * (1.0 / D) - mean_v * mean_v
        rstd = _rsqrt(var_v + 1e-5)
        for i in range(D // L):
            y = (vs[i] - mean_v) * rstd * g_vecs[i] + b_vecs[i]
            out_v[r, pl.ds(L * i, L)] = y
        return c

    lax.fori_loop(0, n, row_body, 0, unroll=8)


def _make_call(batch, seq, n_pairs_rows):
    bat_per_w = batch // NW
    n_chunks = bat_per_w // BCHUNK
    assert n_chunks % 2 == 0
    n_pairs = n_chunks // 2
    nlook = BCHUNK * seq
    mesh = plsc.VectorSubcoreMesh(core_axis_name="c", subcore_axis_name="s")

    @functools.partial(
        pl.kernel,
        mesh=mesh,
        out_type=jax.ShapeDtypeStruct((batch, seq, D), jnp.float32),
        scratch_types=[
            [pltpu.VMEM((BCHUNK * seq,), jnp.int32)] * 4,
            [pltpu.VMEM((BCHUNK * seq, 2 * D), jnp.float32)] * 4,
            pltpu.VMEM((BCHUNK * seq, D), jnp.float32),
            pltpu.VMEM((2 * D,), jnp.float32),
            [pltpu.SemaphoreType.DMA] * 4,
            pltpu.SemaphoreType.DMA,
        ],
        compiler_params=pltpu.CompilerParams(
            needs_layout_passes=False, use_tc_tiling_on_sc=True),
    )
    def call(idp_hbm, table_hbm, gb_hbm, out_hbm,
             idxs, rowss, out_v, gb_v, gsems, osem):
        wid = lax.axis_index("s") * NC + lax.axis_index("c")
        base = wid * bat_per_w
        pltpu.sync_copy(gb_hbm, gb_v)
        g_vecs = [gb_v[pl.ds(L * i, L)] for i in range(D // L)]
        b_vecs = [gb_v[pl.ds(D + L * i, L)] for i in range(D // L)]

        def brow_of(g):
            return pl.multiple_of(base + g * BCHUNK, 4)

        def fire_gather(g, b):
            off = pl.multiple_of((base + g * BCHUNK) * seq, 8)
            pltpu.sync_copy(idp_hbm.at[pl.ds(off, nlook)], idxs[b])
            pltpu.async_copy(table_hbm.at[idxs[b]], rowss[b], gsems[b])

        def wait_gather(b):
            pltpu.make_async_copy(
                table_hbm.at[idxs[b]], rowss[b], gsems[b]).wait()

        def fire_out(g):
            brow = brow_of(g)
            for b in range(BCHUNK):
                pltpu.async_copy(
                    out_v.at[pl.ds(b * seq, seq)], out_hbm.at[brow + b], osem)

        def drain_out():
            for b in range(BCHUNK):
                pltpu.make_async_copy(
                    out_hbm.at[0], out_v.at[pl.ds(b * seq, seq)], osem).wait()

        fire_gather(0, 0)
        fire_gather(1, 1)
        fire_gather(2, 2)
        n_quads = n_chunks // 4

        def quad_body(q, carry):
            for b in range(4):
                g = 4 * q + b
                wait_gather(b)
                if b == 0:
                    @pl.when(q > 0)
                    def _():
                        drain_out()
                else:
                    drain_out()
                _layernorm_chunk(rowss[b], out_v, nlook, g_vecs, b_vecs)
                fire_out(g)
                nb = (b + 3) % 4
                if b == 0:
                    fire_gather(g + 3, nb)
                else:
                    @pl.when(q < n_quads - 1)
                    def _():
                        fire_gather(g + 3, nb)
            return carry

        lax.fori_loop(0, n_quads, quad_body, 0, unroll=False)

        drain_out()

    return call


def kernel(stock_ids, table, gamma, beta):
    batch, seq = stock_ids.shape
    ids = stock_ids.reshape(-1).astype(jnp.int32)
    table2 = jnp.pad(table, ((0, 0), (0, D)))
    gb = jnp.concatenate([gamma, beta]).astype(jnp.float32)
    return _make_call(batch, seq, table2.shape[0])(ids, table2, gb)


# speedup vs baseline: 1.4627x; 1.4627x over previous
import functools

import jax
import jax.numpy as jnp
from jax import lax
from jax.experimental import pallas as pl
from jax.experimental.pallas import tpu as pltpu
from jax.experimental.pallas import tpu_sc as plsc

D = 64
L = 16
NC = 2
NS = 16
NW = NC * NS
BCHUNK = 4


def _shuf(x, idx):
    dn = lax.GatherDimensionNumbers(
        offset_dims=(), collapsed_slice_dims=(0,), start_index_map=(0,))
    return lax.gather(x, idx[:, None], dn, (1,),
                      mode=lax.GatherScatterMode.PROMISE_IN_BOUNDS)


def _allsum(x):
    idx = lax.iota(jnp.int32, L)
    for d in (8, 4, 2, 1):
        x = x + _shuf(x, jnp.bitwise_xor(idx, d))
    return x


def _rsqrt(x):
    bits = plsc.bitcast(x, jnp.int32)
    bits = jnp.int32(0x5F3759DF) - lax.shift_right_logical(bits, 1)
    y = plsc.bitcast(bits, jnp.float32)
    for _ in range(2):
        y = y * (1.5 - 0.5 * x * y * y)
    return y


def _layernorm_chunk(rows_v, out_v, n, g_vecs, b_vecs):
    @plsc.parallel_loop(0, n, unroll=8)
    def row_body(r):
        vs = [rows_v[r, pl.ds(L * i, L)] for i in range(D // L)]
        t = (vs[0] + vs[1]) + (vs[2] + vs[3])
        q = (vs[0] * vs[0] + vs[1] * vs[1]) + (vs[2] * vs[2] + vs[3] * vs[3])
        mean_v = _allsum(t) * (1.0 / D)
        var_v = _allsum(q) * (1.0 / D) - mean_v * mean_v
        rstd = _rsqrt(var_v + 1e-5)
        for i in range(D // L):
            y = (vs[i] - mean_v) * rstd * g_vecs[i] + b_vecs[i]
            out_v[r, pl.ds(L * i, L)] = y


def _make_call(batch, seq, n_pairs_rows):
    bat_per_w = batch // NW
    n_chunks = bat_per_w // BCHUNK
    assert n_chunks % 2 == 0
    n_pairs = n_chunks // 2
    nlook = BCHUNK * seq
    mesh = plsc.VectorSubcoreMesh(core_axis_name="c", subcore_axis_name="s")

    @functools.partial(
        pl.kernel,
        mesh=mesh,
        out_type=jax.ShapeDtypeStruct((batch, seq, D), jnp.float32),
        scratch_types=[
            [pltpu.VMEM((BCHUNK * seq,), jnp.int32)] * 4,
            [pltpu.VMEM((BCHUNK * seq, 2 * D), jnp.float32)] * 4,
            pltpu.VMEM((BCHUNK * seq, D), jnp.float32),
            pltpu.VMEM((2 * D,), jnp.float32),
            [pltpu.SemaphoreType.DMA] * 4,
            pltpu.SemaphoreType.DMA,
        ],
        compiler_params=pltpu.CompilerParams(
            needs_layout_passes=False, use_tc_tiling_on_sc=True),
    )
    def call(idp_hbm, table_hbm, gb_hbm, out_hbm,
             idxs, rowss, out_v, gb_v, gsems, osem):
        wid = lax.axis_index("s") * NC + lax.axis_index("c")
        base = wid * bat_per_w
        pltpu.sync_copy(gb_hbm, gb_v)
        g_vecs = [gb_v[pl.ds(L * i, L)] for i in range(D // L)]
        b_vecs = [gb_v[pl.ds(D + L * i, L)] for i in range(D // L)]

        def brow_of(g):
            return pl.multiple_of(base + g * BCHUNK, 4)

        def fire_gather(g, b):
            off = pl.multiple_of((base + g * BCHUNK) * seq, 8)
            pltpu.sync_copy(idp_hbm.at[pl.ds(off, nlook)], idxs[b])
            pltpu.async_copy(table_hbm.at[idxs[b]], rowss[b], gsems[b])

        def wait_gather(b):
            pltpu.make_async_copy(
                table_hbm.at[idxs[b]], rowss[b], gsems[b]).wait()

        def fire_out(g):
            brow = brow_of(g)
            for b in range(BCHUNK):
                pltpu.async_copy(
                    out_v.at[pl.ds(b * seq, seq)], out_hbm.at[brow + b], osem)

        def drain_out():
            for b in range(BCHUNK):
                pltpu.make_async_copy(
                    out_hbm.at[0], out_v.at[pl.ds(b * seq, seq)], osem).wait()

        fire_gather(0, 0)
        fire_gather(1, 1)
        fire_gather(2, 2)
        n_quads = n_chunks // 4

        def quad_body(q, carry):
            for b in range(4):
                g = 4 * q + b
                wait_gather(b)
                if b == 0:
                    @pl.when(q > 0)
                    def _():
                        drain_out()
                else:
                    drain_out()
                _layernorm_chunk(rowss[b], out_v, nlook, g_vecs, b_vecs)
                fire_out(g)
                nb = (b + 3) % 4
                if b == 0:
                    fire_gather(g + 3, nb)
                else:
                    @pl.when(q < n_quads - 1)
                    def _():
                        fire_gather(g + 3, nb)
            return carry

        lax.fori_loop(0, n_quads, quad_body, 0, unroll=False)

        drain_out()

    return call


def kernel(stock_ids, table, gamma, beta):
    batch, seq = stock_ids.shape
    ids = stock_ids.reshape(-1).astype(jnp.int32)
    table2 = jnp.pad(table, ((0, 0), (0, D)))
    gb = jnp.concatenate([gamma, beta]).astype(jnp.float32)
    return _make_call(batch, seq, table2.shape[0])(ids, table2, gb)
